# baseline (speedup 1.0000x reference)
import functools

import jax
import jax.numpy as jnp
from jax import lax
from jax.experimental import pallas as pl
from jax.experimental.pallas import tpu as pltpu
from jax.experimental.pallas import tpu_sc as plsc

_NW = 32
_V_SC = 25600
_RB = 200


def _sum_body(nblk):
    def body(dist_ref, sum_ref):
        pid = pl.program_id(0)

        @pl.when(pid == 0)
        def _():
            sum_ref[...] = jnp.zeros_like(sum_ref)

        sum_ref[...] += dist_ref[...].sum(1)

    return body


def _epilogue_body(g_ref, sum_ref, part_ref, lpw_ref, y_ref,
                   fp_ref, fd_ref, lq_ref, loss_ref):
    T, B = sum_ref.shape
    g = g_ref[...].reshape(T, B, B)
    lane = lax.broadcasted_iota(jnp.int32, g.shape, 2)
    bidx = lax.broadcasted_iota(jnp.int32, g.shape, 1)
    vals = jnp.where(lane == bidx, g, 0.0).sum(-1)
    sums = sum_ref[...] + part_ref[...].sum(0)
    probs = vals / sums
    fp_ref[...] = probs.T
    fd_ref[...] = probs[T - 1:T, :]
    lq = jnp.log(probs).sum(0, keepdims=True)
    lq_ref[...] = lq
    lp = (1.0 - y_ref[...]) * jnp.log(jnp.float32(1e-8)) + lpw_ref[...]
    d = lq - lp
    loss_ref[...] = jnp.mean(d * d).reshape(1, 1)


def _sc_part(flat2d, rows, T, V, v_tc):
    TB, B = rows.shape[0], flat2d.shape[1]
    per_w = TB // _NW
    vs_w = _V_SC // _NW
    n_chunks = vs_w // _RB
    f32 = jnp.float32
    mesh = plsc.VectorSubcoreMesh(core_axis_name="c", subcore_axis_name="s")

    @functools.partial(
        pl.kernel, mesh=mesh,
        out_type=[
            jax.ShapeDtypeStruct((TB, B), f32),
            jax.ShapeDtypeStruct((_NW, T, B), f32),
        ],
        scratch_types=[
            pltpu.VMEM((per_w,), jnp.int32),
            pltpu.VMEM((per_w, B), f32),
            pltpu.VMEM((T, B), f32),
            pltpu.VMEM((_RB, B), f32),
            pltpu.VMEM((_RB, B), f32),
            pltpu.SemaphoreType.DMA,
            pltpu.SemaphoreType.DMA,
            pltpu.SemaphoreType.DMA,
        ],
    )
    def gk(x_hbm, idx_hbm, out_hbm, part_hbm,
           idx_v, rows_v, acc, buf0, buf1, gsem, sem0, sem1):
        wid = lax.axis_index("s") * 2 + lax.axis_index("c")
        base = wid * per_w

        pltpu.sync_copy(idx_hbm.at[pl.ds(base, per_w)], idx_v)
        pltpu.async_copy(x_hbm.at[idx_v], rows_v, gsem).wait()
        pltpu.sync_copy(rows_v, out_hbm.at[pl.ds(base, per_w)])

        for t in range(T):
            for c in range(B // 16):
                acc[t, pl.ds(c * 16, 16)] = jnp.zeros((16,), f32)

        work = [(t, c) for t in range(T) for c in range(n_chunks)]
        bufs = (buf0, buf1)
        sems = (sem0, sem1)

        def row_base(t, c):
            return t * V + v_tc + wid * vs_w + c * _RB

        handles = {}

        def start(k):
            t, c = work[k]
            handles[k] = pltpu.async_copy(
                x_hbm.at[pl.ds(row_base(t, c), _RB)], bufs[k % 2], sems[k % 2]
            )

        start(0)
        for k in range(len(work)):
            if k + 1 < len(work):
                start(k + 1)
            handles[k].wait()
            t, _ = work[k]
            buf = bufs[k % 2]

            @pl.loop(0, _RB, step=8)
            def _(r):
                for c in range(B // 16):
                    sl = pl.ds(c * 16, 16)
                    s = buf[r, sl]
                    for j in range(1, 8):
                        s = s + buf[r + j, sl]
                    acc[t, sl] += s

        pltpu.async_copy(acc, part_hbm.at[wid], gsem).wait()

    return gk(flat2d, rows)


def kernel(distributions, actions, log_p_world, y):
    T, B, V = distributions.shape
    v_tc = V - _V_SC
    C = 6200
    nblk = v_tc // C
    f32 = jnp.float32

    dvb = jnp.transpose(distributions, (0, 2, 1))
    flat2d = dvb.reshape(T * V, B)

    rows = (actions.astype(jnp.int32)
            + (jnp.arange(T, dtype=jnp.int32) * V)[:, None]).reshape(T * B)
    g, part = _sc_part(flat2d, rows, T, V, v_tc)

    sums = pl.pallas_call(
        _sum_body(nblk),
        grid=(nblk,),
        in_specs=[pl.BlockSpec((T, C, B), lambda i: (0, i, 0))],
        out_specs=pl.BlockSpec((T, B), lambda i: (0, 0)),
        out_shape=jax.ShapeDtypeStruct((T, B), f32),
        compiler_params=pltpu.CompilerParams(
            dimension_semantics=("arbitrary",),
        ),
    )(dvb)

    fp, fd, lq, loss = pl.pallas_call(
        _epilogue_body,
        out_shape=[
            jax.ShapeDtypeStruct((B, T), f32),
            jax.ShapeDtypeStruct((1, B), f32),
            jax.ShapeDtypeStruct((1, B), f32),
            jax.ShapeDtypeStruct((1, 1), f32),
        ],
    )(g, sums, part, log_p_world.reshape(1, B), y.reshape(1, B))

    return fp, fd.reshape(B), lq.reshape(B), loss[0, 0]


# speedup vs baseline: 1.0125x; 1.0125x over previous
import functools

import jax
import jax.numpy as jnp
from jax import lax
from jax.experimental import pallas as pl
from jax.experimental.pallas import tpu as pltpu
from jax.experimental.pallas import tpu_sc as plsc


def _body(nblk):
    def body(dist_ref, g_ref, lpw_ref, y_ref,
             fp_ref, fd_ref, lq_ref, loss_ref, sum_acc):
        pid = pl.program_id(0)
        T, B = sum_acc.shape

        @pl.when(pid == 0)
        def _():
            sum_acc[...] = jnp.zeros_like(sum_acc)

        sum_acc[...] += dist_ref[...].sum(1)

        @pl.when(pid == nblk - 1)
        def _():
            g = g_ref[...].reshape(T, B, B)
            lane = lax.broadcasted_iota(jnp.int32, g.shape, 2)
            bidx = lax.broadcasted_iota(jnp.int32, g.shape, 1)
            vals = jnp.where(lane == bidx, g, 0.0).sum(-1)
            probs = vals / sum_acc[...]
            fp_ref[...] = probs.T
            fd_ref[...] = probs[T - 1:T, :]
            lq = jnp.log(probs).sum(0, keepdims=True)
            lq_ref[...] = lq
            lp = (1.0 - y_ref[...]) * jnp.log(jnp.float32(1e-8)) + lpw_ref[...]
            d = lq - lp
            loss_ref[...] = jnp.mean(d * d).reshape(1, 1)

    return body


def _sc_gather(flat2d, rows):
    TB, B = rows.shape[0], flat2d.shape[1]
    n_workers = 32
    per_w = TB // n_workers
    mesh = plsc.VectorSubcoreMesh(core_axis_name="c", subcore_axis_name="s")

    @functools.partial(
        pl.kernel, mesh=mesh,
        out_type=jax.ShapeDtypeStruct((TB, B), jnp.float32),
        scratch_types=[
            pltpu.VMEM((per_w,), jnp.int32),
            pltpu.VMEM((per_w, B), jnp.float32),
            pltpu.SemaphoreType.DMA,
        ],
    )
    def gk(x_hbm, idx_hbm, out_hbm, idx_v, rows_v, sem):
        wid = lax.axis_index("s") * 2 + lax.axis_index("c")
        base = wid * per_w
        pltpu.sync_copy(idx_hbm.at[pl.ds(base, per_w)], idx_v)
        pltpu.async_copy(x_hbm.at[idx_v], rows_v, sem).wait()
        pltpu.sync_copy(rows_v, out_hbm.at[pl.ds(base, per_w)])

    return gk(flat2d, rows)


def kernel(distributions, actions, log_p_world, y):
    T, B, V = distributions.shape
    C = 5000
    nblk = V // C
    f32 = jnp.float32

    dvb = jnp.transpose(distributions, (0, 2, 1))
    flat2d = dvb.reshape(T * V, B)

    rows = (actions.astype(jnp.int32)
            + (jnp.arange(T, dtype=jnp.int32) * V)[:, None]).reshape(T * B)
    g = _sc_gather(flat2d, rows)

    fp, fd, lq, loss = pl.pallas_call(
        _body(nblk),
        grid=(nblk,),
        in_specs=[
            pl.BlockSpec((T, C, B), lambda i: (0, i, 0)),
            pl.BlockSpec((T * B, B), lambda i: (0, 0)),
            pl.BlockSpec((1, B), lambda i: (0, 0)),
            pl.BlockSpec((1, B), lambda i: (0, 0)),
        ],
        out_specs=[
            pl.BlockSpec((B, T), lambda i: (0, 0)),
            pl.BlockSpec((1, B), lambda i: (0, 0)),
            pl.BlockSpec((1, B), lambda i: (0, 0)),
            pl.BlockSpec((1, 1), lambda i: (0, 0)),
        ],
        out_shape=[
            jax.ShapeDtypeStruct((B, T), f32),
            jax.ShapeDtypeStruct((1, B), f32),
            jax.ShapeDtypeStruct((1, B), f32),
            jax.ShapeDtypeStruct((1, 1), f32),
        ],
        scratch_shapes=[pltpu.VMEM((T, B), f32)],
        compiler_params=pltpu.CompilerParams(
            dimension_semantics=("arbitrary",),
        ),
    )(dvb, g, log_p_world.reshape(1, B), y.reshape(1, B))

    return fp, fd.reshape(B), lq.reshape(B), loss[0, 0]


# speedup vs baseline: 1.3086x; 1.2925x over previous
import jax
import jax.numpy as jnp
from jax.experimental import pallas as pl
from jax.experimental.pallas import tpu as pltpu


def _body(nblk, C):
    def body(dist_ref, act_ref, lpw_ref, y_ref,
             probs_ref, lq_ref, loss_ref,
             sum_acc, val_acc):
        pid = pl.program_id(0)
        T, _, B = dist_ref.shape

        @pl.when(pid == 0)
        def _():
            sum_acc[...] = jnp.zeros_like(sum_acc)
            val_acc[...] = jnp.zeros_like(val_acc)

        x = dist_ref[...]
        act = act_ref[...]
        vidx = jax.lax.broadcasted_iota(jnp.int32, x.shape, 1) + pid * C
        hit = vidx == act[:, None, :]
        sum_acc[...] += x.sum(1)
        val_acc[...] += jnp.where(hit, x, 0.0).sum(1)

        @pl.when(pid == nblk - 1)
        def _():
            probs = val_acc[...] / sum_acc[...]
            probs_ref[...] = probs
            lq = jnp.log(probs).sum(0, keepdims=True)
            lq_ref[...] = lq
            lp = (1.0 - y_ref[...]) * jnp.log(jnp.float32(1e-8)) + lpw_ref[...]
            d = lq - lp
            loss_ref[...] = jnp.mean(d * d).reshape(1, 1)

    return body


def kernel(distributions, actions, log_p_world, y):
    T, B, V = distributions.shape
    C = 5000
    nblk = V // C
    f32 = jnp.float32

    dvb = jnp.transpose(distributions, (0, 2, 1))

    probs, lq, loss = pl.pallas_call(
        _body(nblk, C),
        grid=(nblk,),
        in_specs=[
            pl.BlockSpec((T, C, B), lambda i: (0, i, 0)),
            pl.BlockSpec((T, B), lambda i: (0, 0)),
            pl.BlockSpec((1, B), lambda i: (0, 0)),
            pl.BlockSpec((1, B), lambda i: (0, 0)),
        ],
        out_specs=[
            pl.BlockSpec((T, B), lambda i: (0, 0)),
            pl.BlockSpec((1, B), lambda i: (0, 0)),
            pl.BlockSpec((1, 1), lambda i: (0, 0)),
        ],
        out_shape=[
            jax.ShapeDtypeStruct((T, B), f32),
            jax.ShapeDtypeStruct((1, B), f32),
            jax.ShapeDtypeStruct((1, 1), f32),
        ],
        scratch_shapes=[
            pltpu.VMEM((T, B), f32),
            pltpu.VMEM((T, B), f32),
        ],
        compiler_params=pltpu.CompilerParams(
            dimension_semantics=("arbitrary",),
        ),
    )(dvb, actions, log_p_world.reshape(1, B), y.reshape(1, B))

    return probs.T, probs[T - 1], lq.reshape(B), loss[0, 0]
